# Initial kernel scaffold; baseline (speedup 1.0000x reference)
import jax
import jax.numpy as jnp
from jax.experimental import pallas as pl


def kernel(x, table):
    raise NotImplementedError("write your pallas kernel here")


# speedup vs baseline: 1.3422x; 1.3422x over previous
import functools

import jax
import jax.numpy as jnp
from jax import lax
from jax.experimental import pallas as pl
from jax.experimental.pallas import tpu as pltpu
from jax.experimental.pallas import tpu_sc as plsc

EMBED_DIM = 5120
NUM_ROWS = 4096 * 20
NUM_CORES = 2
NUM_SUBCORES = 16
NUM_WORKERS = NUM_CORES * NUM_SUBCORES
ROWS_PER_WORKER = NUM_ROWS // NUM_WORKERS
CHUNK = 8
NBUF = 2
NUM_CHUNKS = ROWS_PER_WORKER // CHUNK

_mesh = plsc.VectorSubcoreMesh(core_axis_name="c", subcore_axis_name="s")


@functools.partial(
    pl.kernel,
    mesh=_mesh,
    out_type=jax.ShapeDtypeStruct((NUM_ROWS, EMBED_DIM), jnp.float32),
    scratch_types=[
        pltpu.VMEM((ROWS_PER_WORKER,), jnp.int32),
        pltpu.VMEM((NBUF, CHUNK, EMBED_DIM), jnp.float32),
        pltpu.SemaphoreType.DMA((NBUF,)),
        pltpu.SemaphoreType.DMA((NBUF,)),
    ],
)
def _gather_rows(table_hbm, idx_hbm, out_hbm, idx_v, rows_v, gsem, ssem):
    wid = lax.axis_index("s") * NUM_CORES + lax.axis_index("c")
    base = wid * ROWS_PER_WORKER
    pltpu.sync_copy(idx_hbm.at[pl.ds(base, ROWS_PER_WORKER)], idx_v)

    def gather(g, b):
        return pltpu.make_async_copy(
            table_hbm.at[idx_v.at[pl.ds(g * CHUNK, CHUNK)]],
            rows_v.at[b],
            gsem.at[b],
        )

    def scatter(g, b):
        return pltpu.make_async_copy(
            rows_v.at[b],
            out_hbm.at[pl.ds(base + g * CHUNK, CHUNK)],
            ssem.at[b],
        )

    for b in range(NBUF):
        gather(b, b).start()

    def outer(i, carry):
        for b in range(NBUF):
            g = i * NBUF + b
            gather(g, b).wait()
            scatter(g, b).start()
            scatter(g, b).wait()
            nxt = g + NBUF

            @pl.when(nxt < NUM_CHUNKS)
            def _():
                gather(nxt, b).start()

        return carry

    lax.fori_loop(0, NUM_CHUNKS // NBUF, outer, 0)


def kernel(x, table):
    idx = x.reshape(-1)
    out = _gather_rows(table, idx)
    return out.reshape(x.shape + (table.shape[1],))
